# baseline (speedup 1.0000x reference)
import functools

import jax
import jax.numpy as jnp
from jax import lax
from jax.experimental import pallas as pl
from jax.experimental.pallas import tpu as pltpu
from jax.experimental.pallas import tpu_sc as plsc

_N = 512
_K = 1024
_D = 256
_NCAND = 4
_B = _NCAND * _N
_NC = 2
_NS = 16
_NW = _NC * _NS
_BPW = _B // _NW


def _topk_kernel(in_ref, cb_ref, cand_ref):
    xt = jnp.concatenate([in_ref[0], in_ref[1]], axis=1)
    cb = cb_ref[...]
    cb1 = cb.astype(jnp.bfloat16)
    cb2 = (cb - cb1.astype(jnp.float32)).astype(jnp.bfloat16)
    xt1 = xt.astype(jnp.bfloat16)
    xt2 = (xt - xt1.astype(jnp.float32)).astype(jnp.bfloat16)
    cdims = (((1,), (0,)), ((), ()))
    xct = (lax.dot_general(cb1, xt1, dimension_numbers=cdims,
                           preferred_element_type=jnp.float32)
           + lax.dot_general(cb1, xt2, dimension_numbers=cdims,
                             preferred_element_type=jnp.float32)
           + lax.dot_general(cb2, xt1, dimension_numbers=cdims,
                             preferred_element_type=jnp.float32))
    cb2sq = cb * cb
    sq1 = cb2sq.astype(jnp.bfloat16)
    sq2 = (cb2sq - sq1.astype(jnp.float32)).astype(jnp.bfloat16)
    ones = jnp.ones((_D, 1), jnp.bfloat16)
    cnorm = (lax.dot_general(sq1, ones, dimension_numbers=cdims,
                             preferred_element_type=jnp.float32)
             + lax.dot_general(sq2, ones, dimension_numbers=cdims,
                               preferred_element_type=jnp.float32))
    st = cnorm - 2.0 * xct
    riota = lax.broadcasted_iota(jnp.int32, (_K, _N), 0)
    qs = lax.convert_element_type(st * jnp.float32(1 << 20), jnp.int32)
    packed = ((qs + jnp.int32(1 << 20)) << 10) | riota
    rows = []
    for _ in range(_NCAND):
        mp = jnp.min(packed, axis=0, keepdims=True)
        rows.append(jnp.bitwise_and(mp, jnp.int32(_K - 1)))
        packed = jnp.where(packed == mp, jnp.int32(0x7fffffff), packed)
    cand_ref[...] = jnp.concatenate(rows, axis=0)


def _sc_gather_body(cb_hbm, idx_hbm, out_hbm, idx_v, rows_v, sem):
    wid = lax.axis_index("s") * _NC + lax.axis_index("c")
    base = wid * _BPW
    pltpu.sync_copy(idx_hbm.at[pl.ds(base, _BPW)], idx_v)
    pltpu.async_copy(cb_hbm.at[idx_v], rows_v, sem).wait()
    pltpu.sync_copy(rows_v, out_hbm.at[pl.ds(base, _BPW)])


@functools.cache
def _sc_gather():
    mesh = plsc.VectorSubcoreMesh(
        core_axis_name="c", subcore_axis_name="s", num_cores=_NC,
        num_subcores=_NS)
    return pl.kernel(
        _sc_gather_body,
        out_type=jax.ShapeDtypeStruct((_B, _D), jnp.float32),
        mesh=mesh,
        scratch_types=[
            pltpu.VMEM((_BPW,), jnp.int32),
            pltpu.VMEM((_BPW, _D), jnp.float32),
            pltpu.SemaphoreType.DMA,
        ],
    )


def _refined_dist(xt, rowt):
    sq = (xt - rowt) * (xt - rowt)
    totals = []
    for c in range(2):
        r = sq[128 * c:128 * (c + 1), :].reshape(16, 8, _N)
        p = r[0]
        for k in range(1, 16):
            p = p + r[k]
        a = ((p[0:1] + p[4:5]) + (p[2:3] + p[6:7])) + (
            (p[1:2] + p[5:6]) + (p[3:4] + p[7:8]))
        totals.append(a)
    return (totals[0] + totals[1]) * jnp.float32(1.0 / _D)


def _refine_kernel(in_ref, rows_ref, cand_ref, loss_ref, q_ref, idx_ref):
    xt = jnp.concatenate([in_ref[0], in_ref[1]], axis=1)
    best_d = None
    best_i = None
    best_qt = None
    for c in range(_NCAND):
        rowt = jnp.transpose(rows_ref[c], (1, 0))
        i_c = cand_ref[c:c + 1, :]
        d = _refined_dist(xt, rowt)
        if best_d is None:
            best_d, best_i, best_qt = d, i_c, rowt
        else:
            lt = (d < best_d) | ((d == best_d) & (i_c < best_i))
            best_d = jnp.where(lt, d, best_d)
            best_i = jnp.where(lt, i_c, best_i)
            best_qt = jnp.where(lt, rowt, best_qt)
    diff = best_qt - xt
    m2 = jnp.sum(diff * diff) * jnp.float32(1.0 / (_N * _D))
    loss_ref[...] = (m2 + jnp.float32(0.25) * m2)[None, None]
    qst = xt + (best_qt - xt)
    q_ref[...] = jnp.stack([qst[:, :_N // 2], qst[:, _N // 2:]], axis=0)
    idx_ref[...] = best_i


@jax.jit
def kernel(inputs, codebook):
    flat = inputs.reshape(2, _D, 256)
    cand = pl.pallas_call(
        _topk_kernel,
        out_shape=jax.ShapeDtypeStruct((_NCAND, _N), jnp.int32),
    )(flat, codebook)
    rows = _sc_gather()(codebook, cand.reshape(_B))
    loss, q, idx = pl.pallas_call(
        _refine_kernel,
        out_shape=(
            jax.ShapeDtypeStruct((1, 1), jnp.float32),
            jax.ShapeDtypeStruct((2, _D, 256), jnp.float32),
            jax.ShapeDtypeStruct((1, _N), jnp.int32),
        ),
    )(flat, rows.reshape(_NCAND, _N, _D), cand)
    quantized = q.reshape(2, _D, 16, 16)
    return loss.reshape(()), quantized, idx.reshape(2, 256)
